# Initial kernel scaffold; baseline (speedup 1.0000x reference)
import jax
import jax.numpy as jnp
from jax.experimental import pallas as pl


def kernel(edge_index, x, w, W_weight, W_bias):
    raise NotImplementedError("write your pallas kernel here")


# speedup vs baseline: 2.4189x; 2.4189x over previous
import functools

import jax
import jax.numpy as jnp
from jax import lax
from jax.experimental import pallas as pl
from jax.experimental.pallas import tpu as pltpu
from jax.experimental.pallas import tpu_sc as plsc

N_NODES = 10000
N_EDGES = 320000
D_FEAT = 128
OUT_DIM = 64

NC = 2
NS = 16
NW = NC * NS
L = 16

DAUG = 144
N_PAD = 10240
CHUNK = 80
NCHUNK = 128
EPW = CHUNK * NCHUNK
E_PAD = NW * EPW
RPT = N_PAD // NS


def _sc_segment_kernel(xaug_hbm, idx_hbm, wrows_hbm,
                       msg_out, deg_out, wsum_out,
                       idx0_v, idx1_v, rows0_v, rows1_v, wbuf_v,
                       acc_s, accw_s, sem0, sem1):
    c = lax.axis_index("c")
    s = lax.axis_index("s")
    wid = c * NS + s

    zeros16 = jnp.zeros((L,), jnp.float32)

    def _zero_bufs(i, _):
        for k in range(DAUG // L):
            rows0_v[i, pl.ds(k * L, L)] = zeros16
        wbuf_v[i, pl.ds(0, L)] = zeros16
        return 0

    lax.fori_loop(0, CHUNK, _zero_bufs, 0)

    r0 = s * RPT
    for k in range(RPT // CHUNK):
        pltpu.sync_copy(rows0_v, acc_s.at[pl.ds(r0 + k * CHUNK, CHUNK)])
        pltpu.sync_copy(wbuf_v, accw_s.at[pl.ds(r0 + k * CHUNK, CHUNK)])
    plsc.subcore_barrier()

    def _commit(j, idx_v, rows_v):
        pltpu.sync_copy(rows_v, acc_s.at[idx_v.at[1]], add=True)
        pltpu.sync_copy(wrows_hbm.at[wid, pl.ds(j * CHUNK, CHUNK)], wbuf_v)
        pltpu.sync_copy(wbuf_v, accw_s.at[idx_v.at[0]], add=True)

    def _body(i, _):
        j0 = 2 * i
        j1 = 2 * i + 1
        pltpu.sync_copy(idx_hbm.at[wid, j0], idx0_v)
        g0 = pltpu.async_copy(xaug_hbm.at[idx0_v.at[0]], rows0_v, sem0)
        pltpu.sync_copy(idx_hbm.at[wid, j1], idx1_v)
        g1 = pltpu.async_copy(xaug_hbm.at[idx1_v.at[0]], rows1_v, sem1)
        g0.wait()
        _commit(j0, idx0_v, rows0_v)
        g1.wait()
        _commit(j1, idx1_v, rows1_v)
        return 0

    lax.fori_loop(0, NCHUNK // 2, _body, 0)
    plsc.subcore_barrier()

    row0 = c * N_PAD + r0
    pltpu.sync_copy(acc_s.at[pl.ds(r0, RPT), pl.ds(0, D_FEAT)],
                    msg_out.at[pl.ds(row0, RPT)])
    pltpu.sync_copy(acc_s.at[pl.ds(r0, RPT), pl.ds(D_FEAT, L)],
                    deg_out.at[pl.ds(row0, RPT)])
    pltpu.sync_copy(accw_s.at[pl.ds(r0, RPT)],
                    wsum_out.at[pl.ds(row0, RPT)])


def _sc_aggregate(x_aug, idx_r, w_rows):
    mesh = plsc.VectorSubcoreMesh(core_axis_name="c", subcore_axis_name="s")
    f32 = jnp.float32
    return pl.kernel(
        _sc_segment_kernel,
        out_type=(
            jax.ShapeDtypeStruct((NC * N_PAD, D_FEAT), f32),
            jax.ShapeDtypeStruct((NC * N_PAD, L), f32),
            jax.ShapeDtypeStruct((NC * N_PAD, L), f32),
        ),
        mesh=mesh,
        compiler_params=pltpu.CompilerParams(use_tc_tiling_on_sc=False),
        scratch_types=[
            pltpu.VMEM((2, CHUNK), jnp.int32),
            pltpu.VMEM((2, CHUNK), jnp.int32),
            pltpu.VMEM((CHUNK, DAUG), f32),
            pltpu.VMEM((CHUNK, DAUG), f32),
            pltpu.VMEM((CHUNK, L), f32),
            pltpu.VMEM_SHARED((N_PAD, DAUG), f32),
            pltpu.VMEM_SHARED((N_PAD, L), f32),
            pltpu.SemaphoreType.DMA,
            pltpu.SemaphoreType.DMA,
        ],
    )(x_aug, idx_r, w_rows)


TC_BLK = 1024
TC_GRID = N_PAD // TC_BLK


def _tc_finish_kernel(xaug_ref, msg_ref, deg_ref, wsum_ref,
                      w1t_ref, w2t_ref, bias_ref,
                      h_ref, reg_ref, vec_acc, s1_acc):
    i = pl.program_id(0)

    @pl.when(i == 0)
    def _init():
        vec_acc[...] = jnp.zeros_like(vec_acc)
        s1_acc[0] = 0.0

    xa = xaug_ref[...]
    x = xa[:, :D_FEAT]
    msg = msg_ref[0] + msg_ref[1]
    deg = (deg_ref[0] + deg_ref[1])[:, 0]
    y = msg / jnp.maximum(deg, 1.0)[:, None]
    h = (jnp.dot(x, w1t_ref[...], preferred_element_type=jnp.float32)
         + jnp.dot(y, w2t_ref[...], preferred_element_type=jnp.float32)
         + bias_ref[...])
    h_ref[...] = h

    a = (wsum_ref[0] + wsum_ref[1])[:, 0] * (1.0 / N_NODES)
    vec_acc[0, :] += jnp.sum(y, axis=0)
    vec_acc[1, :] += jnp.sum(a[:, None] * x, axis=0)
    s1_acc[0] += jnp.sum((a * a) * jnp.sum(x * x, axis=1))

    @pl.when(i == TC_GRID - 1)
    def _fin():
        m = vec_acc[0, :] * (1.0 / N_NODES)
        v = vec_acc[1, :]
        reg = (s1_acc[0] - 2.0 * jnp.sum(v * m)
               + N_NODES * jnp.sum(m * m)) / (N_NODES * OUT_DIM)
        reg_ref[...] = jnp.reshape(reg, (1, 1))


def _tc_finish(x_aug, msg2, deg2, wsum2, w1t, w2t, bias2d):
    f32 = jnp.float32
    return pl.pallas_call(
        _tc_finish_kernel,
        grid=(TC_GRID,),
        in_specs=[
            pl.BlockSpec((TC_BLK, DAUG), lambda i: (i, 0)),
            pl.BlockSpec((NC, TC_BLK, D_FEAT), lambda i: (0, i, 0)),
            pl.BlockSpec((NC, TC_BLK, L), lambda i: (0, i, 0)),
            pl.BlockSpec((NC, TC_BLK, L), lambda i: (0, i, 0)),
            pl.BlockSpec((D_FEAT, OUT_DIM), lambda i: (0, 0)),
            pl.BlockSpec((D_FEAT, OUT_DIM), lambda i: (0, 0)),
            pl.BlockSpec((1, OUT_DIM), lambda i: (0, 0)),
        ],
        out_specs=[
            pl.BlockSpec((TC_BLK, OUT_DIM), lambda i: (i, 0)),
            pl.BlockSpec((1, 1), lambda i: (0, 0)),
        ],
        out_shape=[
            jax.ShapeDtypeStruct((N_PAD, OUT_DIM), f32),
            jax.ShapeDtypeStruct((1, 1), f32),
        ],
        scratch_shapes=[
            pltpu.VMEM((2, D_FEAT), f32),
            pltpu.SMEM((1,), f32),
        ],
    )(x_aug, msg2, deg2, wsum2, w1t, w2t, bias2d)


def kernel(edge_index, x, w, W_weight, W_bias):
    src = edge_index[0]
    dst = edge_index[1]

    pad = E_PAD - N_EDGES
    src_p = jnp.concatenate([src, jnp.full((pad,), N_NODES, jnp.int32)])
    dst_p = jnp.concatenate([dst, jnp.full((pad,), N_NODES, jnp.int32)])
    w_p = jnp.concatenate([w, jnp.zeros((pad,), jnp.float32)])
    idx_r = jnp.stack([src_p.reshape(NW, NCHUNK, CHUNK),
                       dst_p.reshape(NW, NCHUNK, CHUNK)], axis=2)
    w_rows = jnp.pad(w_p[:, None], ((0, 0), (0, L - 1))).reshape(NW, EPW, L)

    x_aug = jnp.zeros((N_PAD, DAUG), jnp.float32)
    x_aug = x_aug.at[:N_NODES, :D_FEAT].set(x)
    x_aug = x_aug.at[:N_NODES, D_FEAT].set(1.0)

    msg2, deg2, wsum2 = _sc_aggregate(x_aug, idx_r, w_rows)
    msg2 = msg2.reshape(NC, N_PAD, D_FEAT)
    deg2 = deg2.reshape(NC, N_PAD, L)
    wsum2 = wsum2.reshape(NC, N_PAD, L)

    w1t = W_weight[:, :D_FEAT].T
    w2t = W_weight[:, D_FEAT:].T
    bias2d = W_bias[None, :]

    h_pad, reg = _tc_finish(x_aug, msg2, deg2, wsum2, w1t, w2t, bias2d)
    return (h_pad[:N_NODES], reg[0, 0])
